# baseline (speedup 1.0000x reference)
import functools

import jax
import jax.numpy as jnp
from jax.experimental import pallas as pl
from jax.experimental.pallas import tpu as pltpu

NUM_GROUP = 512
GROUP_SIZE = 32
IN_CH = 3
ENC_CH = 384


def _fps(xyz, n_samples):
    B, N, _ = xyz.shape
    dists0 = jnp.full((B, N), 1e10, dtype=xyz.dtype)
    far0 = jnp.zeros((B,), dtype=jnp.int32)

    def step(carry, _):
        dists, farthest = carry
        centroid = jnp.take_along_axis(xyz, farthest[:, None, None].astype(jnp.int32), axis=1)
        d = jnp.sum((xyz - centroid) ** 2, axis=-1)
        dists = jnp.minimum(dists, d)
        nxt = jnp.argmax(dists, axis=1).astype(jnp.int32)
        return (dists, nxt), farthest

    (_, _), idxs = jax.lax.scan(step, (dists0, far0), None, length=n_samples)
    return jnp.transpose(idxs)


def _fps_kernel(xt_ref, center_ref, dists_ref):
    _, B, N = xt_ref.shape
    G = center_ref.shape[2]
    x = xt_ref[0]
    y = xt_ref[1]
    z = xt_ref[2]
    dists_ref[...] = jnp.full((B, N), 1e10, jnp.float32)
    iota_n = jax.lax.broadcasted_iota(jnp.int32, (B, N), 1)
    iota_g = jax.lax.broadcasted_iota(jnp.int32, (B, G), 1)

    def body(t, far):
        oh = iota_n == far
        cx = jnp.sum(jnp.where(oh, x, 0.0), axis=1, keepdims=True)
        cy = jnp.sum(jnp.where(oh, y, 0.0), axis=1, keepdims=True)
        cz = jnp.sum(jnp.where(oh, z, 0.0), axis=1, keepdims=True)
        ohg = iota_g == t
        center_ref[0] = jnp.where(ohg, cx, center_ref[0])
        center_ref[1] = jnp.where(ohg, cy, center_ref[1])
        center_ref[2] = jnp.where(ohg, cz, center_ref[2])
        dx = x - cx
        dy = y - cy
        dz = z - cz
        d = dx * dx + dy * dy + dz * dz
        nd = jnp.minimum(dists_ref[...], d)
        dists_ref[...] = nd
        m = jnp.max(nd, axis=1, keepdims=True)
        far2 = jnp.min(jnp.where(nd == m, iota_n, N), axis=1, keepdims=True)
        return far2

    far0 = jnp.zeros((B, 1), jnp.int32)
    jax.lax.fori_loop(0, G, body, far0)


def _fps_centers(xyz, n_samples, interpret=False):
    B, N, _ = xyz.shape
    xt = jnp.transpose(xyz, (2, 0, 1))
    center3 = pl.pallas_call(
        _fps_kernel,
        out_shape=jax.ShapeDtypeStruct((3, B, n_samples), jnp.float32),
        scratch_shapes=[pltpu.VMEM((B, N), jnp.float32)],
        interpret=interpret,
    )(xt)
    return jnp.transpose(center3, (1, 2, 0))


_GBLK = 128


def _knn_kernel(din_ref, xt_ref, c_ref, nbh_ref, d_ref):
    N = din_ref.shape[2]
    K = nbh_ref.shape[1]
    c = c_ref[0]
    d_ref[...] = din_ref[0]
    iota_n = jax.lax.broadcasted_iota(jnp.int32, (c.shape[0], N), 1)
    x = xt_ref[0, 0:1]
    y = xt_ref[0, 1:2]
    z = xt_ref[0, 2:3]
    for k in range(K):
        dcur = d_ref[...]
        m = jnp.min(dcur, axis=1, keepdims=True)
        ii = jnp.min(jnp.where(dcur <= m, iota_n, N), axis=1, keepdims=True)
        sel = iota_n == ii
        gx = jnp.sum(jnp.where(sel, x, 0.0), axis=1, keepdims=True)
        gy = jnp.sum(jnp.where(sel, y, 0.0), axis=1, keepdims=True)
        gz = jnp.sum(jnp.where(sel, z, 0.0), axis=1, keepdims=True)
        nbk = jnp.concatenate([gx, gy, gz], axis=1)
        nbh_ref[0, k] = nbk - c
        d_ref[...] = jnp.where(sel, 1e30, dcur)


def _knn_neighborhood(xyz, center, interpret=False):
    B, N, _ = xyz.shape
    G = center.shape[1]
    K = GROUP_SIZE
    dist = _square_distance(center, xyz)
    xt = jnp.transpose(xyz, (0, 2, 1))
    nbh = pl.pallas_call(
        _knn_kernel,
        grid=(B, G // _GBLK),
        in_specs=[
            pl.BlockSpec((1, _GBLK, N), lambda b, g: (b, g, 0)),
            pl.BlockSpec((1, 3, N), lambda b, g: (b, 0, 0)),
            pl.BlockSpec((1, _GBLK, 3), lambda b, g: (b, g, 0)),
        ],
        out_specs=pl.BlockSpec((1, K, _GBLK, 3), lambda b, g: (b, 0, g, 0)),
        out_shape=jax.ShapeDtypeStruct((B, K, G, 3), jnp.float32),
        scratch_shapes=[pltpu.VMEM((_GBLK, N), jnp.float32)],
        compiler_params=pltpu.CompilerParams(
            dimension_semantics=("parallel", "parallel")),
        interpret=interpret,
    )(dist, xt, center)
    return jnp.transpose(nbh, (0, 2, 1, 3))


def _xstats_kernel(x3_ref, st_ref):
    a = x3_ref[0:1]
    b = x3_ref[1:2]
    c = x3_ref[2:3]
    def s(v):
        return jnp.sum(v, axis=1, keepdims=True)

    vals = [
        s(a), s(b), s(c),
        s(a * a), s(a * b), s(a * c),
        s(b * b), s(b * c), s(c * c),
        jnp.zeros((1, 7), jnp.float32),
    ]
    st_ref[...] = jnp.concatenate(vals, axis=1)


_SBLK = 4096


def _enc1_kernel(x_ref, w1_ref, b1_ref, w2_ref, b2_ref, w3_ref, b3_ref,
                 f3_ref, st_ref):
    S = x_ref.shape[0]
    K = GROUP_SIZE
    f1 = jnp.dot(x_ref[...], w1_ref[...], preferred_element_type=jnp.float32,
                 precision=jax.lax.Precision.HIGHEST)
    f1 = jnp.maximum(f1 + b1_ref[...], 0.0)
    f2 = jnp.dot(f1, w2_ref[...], preferred_element_type=jnp.float32,
                 precision=jax.lax.Precision.HIGHEST)
    f2 = f2 + b2_ref[...]
    f2g = f2.reshape(S // K, K, f2.shape[1])
    fg = jnp.max(f2g, axis=1, keepdims=True)
    fgb = jnp.broadcast_to(fg, f2g.shape).reshape(S, f2.shape[1])
    h = jnp.concatenate([fgb, f2], axis=1)
    f3 = jnp.dot(h, w3_ref[...], preferred_element_type=jnp.float32,
                 precision=jax.lax.Precision.HIGHEST)
    f3 = f3 + b3_ref[...]
    f3_ref[...] = f3
    st_ref[0, 0:1] = jnp.sum(f3, axis=0, keepdims=True)
    st_ref[0, 1:2] = jnp.sum(f3 * f3, axis=0, keepdims=True)


def _enc2_kernel(f3_ref, sc_ref, sh_ref, w4_ref, b4_ref, tok_ref):
    S = f3_ref.shape[0]
    K = GROUP_SIZE
    f3n = jnp.maximum(f3_ref[...] * sc_ref[...] + sh_ref[...], 0.0)
    f4 = jnp.dot(f3n, w4_ref[...], preferred_element_type=jnp.float32,
                 precision=jax.lax.Precision.HIGHEST)
    f4 = f4 + b4_ref[...]
    tok_ref[...] = jnp.max(f4.reshape(S // K, K, f4.shape[1]), axis=1)


def _encoder_pallas(neighborhood, W1, b1, g1, be1, W2, b2, W3, b3, g3, be3,
                    W4, b4, interpret=False):
    bs, g, n, _ = neighborhood.shape
    S = bs * g * n
    eps = 1e-5
    X = neighborhood.reshape(S, 3)
    x3 = jnp.transpose(X, (1, 0))
    st = pl.pallas_call(
        _xstats_kernel,
        out_shape=jax.ShapeDtypeStruct((1, 16), jnp.float32),
        interpret=interpret,
    )(x3)[0]
    mean_x = st[0:3] / S
    m2 = jnp.array([[st[3], st[4], st[5]],
                    [st[4], st[6], st[7]],
                    [st[5], st[7], st[8]]]) / S
    cov = m2 - mean_x[:, None] * mean_x[None, :]
    mean1 = W1 @ mean_x + b1
    var1 = jnp.einsum('ci,ij,cj->c', W1, cov, W1)
    scale1 = g1 / jnp.sqrt(var1 + eps)
    bias1 = b1 * scale1 + (be1 - mean1 * scale1)
    W1p = W1.T * scale1[None, :]

    nblk = S // _SBLK
    f3, st3 = pl.pallas_call(
        _enc1_kernel,
        grid=(nblk,),
        in_specs=[
            pl.BlockSpec((_SBLK, 3), lambda i: (i, 0)),
            pl.BlockSpec((3, 128), lambda i: (0, 0)),
            pl.BlockSpec((1, 128), lambda i: (0, 0)),
            pl.BlockSpec((128, 256), lambda i: (0, 0)),
            pl.BlockSpec((1, 256), lambda i: (0, 0)),
            pl.BlockSpec((512, 512), lambda i: (0, 0)),
            pl.BlockSpec((1, 512), lambda i: (0, 0)),
        ],
        out_specs=[
            pl.BlockSpec((_SBLK, 512), lambda i: (i, 0)),
            pl.BlockSpec((1, 2, 512), lambda i: (i, 0, 0)),
        ],
        out_shape=[
            jax.ShapeDtypeStruct((S, 512), jnp.float32),
            jax.ShapeDtypeStruct((nblk, 2, 512), jnp.float32),
        ],
        compiler_params=pltpu.CompilerParams(
            dimension_semantics=("parallel",)),
        interpret=interpret,
    )(X, W1p, bias1.reshape(1, 128), W2.T, b2.reshape(1, 256),
      W3.T, b3.reshape(1, 512))
    ssum = jnp.sum(st3, axis=0)
    mean3 = ssum[0] / S
    var3 = ssum[1] / S - mean3 * mean3
    scale3 = g3 / jnp.sqrt(var3 + eps)
    shift3 = be3 - mean3 * scale3

    tokens = pl.pallas_call(
        _enc2_kernel,
        grid=(nblk,),
        in_specs=[
            pl.BlockSpec((_SBLK, 512), lambda i: (i, 0)),
            pl.BlockSpec((1, 512), lambda i: (0, 0)),
            pl.BlockSpec((1, 512), lambda i: (0, 0)),
            pl.BlockSpec((512, ENC_CH), lambda i: (0, 0)),
            pl.BlockSpec((1, ENC_CH), lambda i: (0, 0)),
        ],
        out_specs=pl.BlockSpec((_SBLK // GROUP_SIZE, ENC_CH), lambda i: (i, 0)),
        out_shape=jax.ShapeDtypeStruct((S // GROUP_SIZE, ENC_CH), jnp.float32),
        compiler_params=pltpu.CompilerParams(
            dimension_semantics=("parallel",)),
        interpret=interpret,
    )(f3, scale3.reshape(1, 512), shift3.reshape(1, 512),
      W4.T, b4.reshape(1, ENC_CH))
    return tokens.reshape(bs, g, ENC_CH)


def _index_points(points, idx):
    return jax.vmap(lambda p, i: p[i])(points, idx)


def _square_distance(src, dst):
    d = -2.0 * jnp.einsum('bsc,bnc->bsn', src, dst)
    d = d + jnp.sum(src ** 2, -1)[:, :, None]
    d = d + jnp.sum(dst ** 2, -1)[:, None, :]
    return d


def _conv1(x, W, b):
    return jnp.einsum('oi,bik->bok', W, x) + b[None, :, None]


def _batchnorm(x, gamma, beta, eps=1e-5):
    mean = jnp.mean(x, axis=(0, 2), keepdims=True)
    var = jnp.var(x, axis=(0, 2), keepdims=True)
    xn = (x - mean) / jnp.sqrt(var + eps)
    return gamma[None, :, None] * xn + beta[None, :, None]


def _sub_kernel(nb_ref, c_ref, o_ref):
    o_ref[...] = nb_ref[...] - c_ref[...]


def _encoder(neighborhood, W1, b1, g1, be1, W2, b2, W3, b3, g3, be3, W4, b4):
    bs, g, n, _ = neighborhood.shape
    pg = neighborhood.reshape(bs * g, n, IN_CH).transpose(0, 2, 1)
    f = _conv1(pg, W1, b1)
    f = jax.nn.relu(_batchnorm(f, g1, be1))
    f = _conv1(f, W2, b2)
    fg = jnp.max(f, axis=2, keepdims=True)
    f = jnp.concatenate([jnp.broadcast_to(fg, (bs * g, 256, n)), f], axis=1)
    f = _conv1(f, W3, b3)
    f = jax.nn.relu(_batchnorm(f, g3, be3))
    f = _conv1(f, W4, b4)
    fg = jnp.max(f, axis=2)
    return fg.reshape(bs, g, ENC_CH)


def kernel(xyz, W1, b1, g1, be1, W2, b2, W3, b3, g3, be3, W4, b4):
    B, N, _ = xyz.shape
    center = _fps_centers(xyz, NUM_GROUP)
    neighborhood = _knn_neighborhood(xyz, center)
    tokens = _encoder_pallas(neighborhood, W1, b1, g1, be1, W2, b2,
                             W3, b3, g3, be3, W4, b4)
    return tokens


# speedup vs baseline: 1.2740x; 1.2740x over previous
import functools

import jax
import jax.numpy as jnp
from jax.experimental import pallas as pl
from jax.experimental.pallas import tpu as pltpu

NUM_GROUP = 512
GROUP_SIZE = 32
IN_CH = 3
ENC_CH = 384


def _fps(xyz, n_samples):
    B, N, _ = xyz.shape
    dists0 = jnp.full((B, N), 1e10, dtype=xyz.dtype)
    far0 = jnp.zeros((B,), dtype=jnp.int32)

    def step(carry, _):
        dists, farthest = carry
        centroid = jnp.take_along_axis(xyz, farthest[:, None, None].astype(jnp.int32), axis=1)
        d = jnp.sum((xyz - centroid) ** 2, axis=-1)
        dists = jnp.minimum(dists, d)
        nxt = jnp.argmax(dists, axis=1).astype(jnp.int32)
        return (dists, nxt), farthest

    (_, _), idxs = jax.lax.scan(step, (dists0, far0), None, length=n_samples)
    return jnp.transpose(idxs)


def _fps_kernel(xt_ref, center_ref, dists_ref):
    _, B, N = xt_ref.shape
    G = center_ref.shape[2]
    x = xt_ref[0]
    y = xt_ref[1]
    z = xt_ref[2]
    dists_ref[...] = jnp.full((B, N), 1e10, jnp.float32)
    iota_n = jax.lax.broadcasted_iota(jnp.int32, (B, N), 1)
    iota_g = jax.lax.broadcasted_iota(jnp.int32, (B, G), 1)

    def body(t, far):
        oh = iota_n == far
        cx = jnp.sum(jnp.where(oh, x, 0.0), axis=1, keepdims=True)
        cy = jnp.sum(jnp.where(oh, y, 0.0), axis=1, keepdims=True)
        cz = jnp.sum(jnp.where(oh, z, 0.0), axis=1, keepdims=True)
        ohg = iota_g == t
        center_ref[0] = jnp.where(ohg, cx, center_ref[0])
        center_ref[1] = jnp.where(ohg, cy, center_ref[1])
        center_ref[2] = jnp.where(ohg, cz, center_ref[2])
        dx = x - cx
        dy = y - cy
        dz = z - cz
        d = dx * dx + dy * dy + dz * dz
        nd = jnp.minimum(dists_ref[...], d)
        dists_ref[...] = nd
        m = jnp.max(nd, axis=1, keepdims=True)
        far2 = jnp.min(jnp.where(nd == m, iota_n, N), axis=1, keepdims=True)
        return far2

    far0 = jnp.zeros((B, 1), jnp.int32)
    jax.lax.fori_loop(0, G, body, far0)


def _fps_centers(xyz, n_samples, interpret=False):
    B, N, _ = xyz.shape
    xt = jnp.transpose(xyz, (2, 0, 1))
    center3 = pl.pallas_call(
        _fps_kernel,
        out_shape=jax.ShapeDtypeStruct((3, B, n_samples), jnp.float32),
        scratch_shapes=[pltpu.VMEM((B, N), jnp.float32)],
        interpret=interpret,
    )(xt)
    return jnp.transpose(center3, (1, 2, 0))


_GBLK = 128


def _knn_kernel(din_ref, xt_ref, c_ref, nbh_ref, d_ref):
    N = din_ref.shape[2]
    K = nbh_ref.shape[1]
    c = c_ref[0]
    d_ref[...] = din_ref[0]
    iota_n = jax.lax.broadcasted_iota(jnp.int32, (c.shape[0], N), 1)
    x = xt_ref[0, 0:1]
    y = xt_ref[0, 1:2]
    z = xt_ref[0, 2:3]
    for k in range(K):
        dcur = d_ref[...]
        m = jnp.min(dcur, axis=1, keepdims=True)
        ii = jnp.min(jnp.where(dcur <= m, iota_n, N), axis=1, keepdims=True)
        sel = iota_n == ii
        gx = jnp.sum(jnp.where(sel, x, 0.0), axis=1, keepdims=True)
        gy = jnp.sum(jnp.where(sel, y, 0.0), axis=1, keepdims=True)
        gz = jnp.sum(jnp.where(sel, z, 0.0), axis=1, keepdims=True)
        nbk = jnp.concatenate([gx, gy, gz], axis=1)
        nbh_ref[0, k] = nbk - c
        d_ref[...] = jnp.where(sel, 1e30, dcur)


def _knn_neighborhood(xyz, center, interpret=False):
    B, N, _ = xyz.shape
    G = center.shape[1]
    K = GROUP_SIZE
    dist = _square_distance(center, xyz)
    xt = jnp.transpose(xyz, (0, 2, 1))
    nbh = pl.pallas_call(
        _knn_kernel,
        grid=(B, G // _GBLK),
        in_specs=[
            pl.BlockSpec((1, _GBLK, N), lambda b, g: (b, g, 0)),
            pl.BlockSpec((1, 3, N), lambda b, g: (b, 0, 0)),
            pl.BlockSpec((1, _GBLK, 3), lambda b, g: (b, g, 0)),
        ],
        out_specs=pl.BlockSpec((1, K, _GBLK, 3), lambda b, g: (b, 0, g, 0)),
        out_shape=jax.ShapeDtypeStruct((B, K, G, 3), jnp.float32),
        scratch_shapes=[pltpu.VMEM((_GBLK, N), jnp.float32)],
        compiler_params=pltpu.CompilerParams(
            dimension_semantics=("parallel", "parallel")),
        interpret=interpret,
    )(dist, xt, center)
    return jnp.transpose(nbh, (0, 2, 1, 3))


def _xstats_kernel(x3_ref, st_ref):
    a = x3_ref[0:1]
    b = x3_ref[1:2]
    c = x3_ref[2:3]
    def s(v):
        return jnp.sum(v, axis=1, keepdims=True)

    vals = [
        s(a), s(b), s(c),
        s(a * a), s(a * b), s(a * c),
        s(b * b), s(b * c), s(c * c),
        jnp.zeros((1, 7), jnp.float32),
    ]
    st_ref[...] = jnp.concatenate(vals, axis=1)


_SBLK = 4096


def _enc1_kernel(x_ref, w1_ref, b1_ref, w2_ref, b2_ref, w3_ref, b3_ref,
                 f3_ref, st_ref):
    S = x_ref.shape[0]
    K = GROUP_SIZE
    f1 = jnp.dot(x_ref[...], w1_ref[...], preferred_element_type=jnp.float32)
    f1 = jnp.maximum(f1 + b1_ref[...], 0.0)
    f2 = jnp.dot(f1, w2_ref[...], preferred_element_type=jnp.float32)
    f2 = f2 + b2_ref[...]
    f2g = f2.reshape(S // K, K, f2.shape[1])
    fg = jnp.max(f2g, axis=1, keepdims=True)
    fgb = jnp.broadcast_to(fg, f2g.shape).reshape(S, f2.shape[1])
    h = jnp.concatenate([fgb, f2], axis=1)
    f3 = jnp.dot(h, w3_ref[...], preferred_element_type=jnp.float32)
    f3 = f3 + b3_ref[...]
    f3_ref[...] = f3
    st_ref[0, 0:1] = jnp.sum(f3, axis=0, keepdims=True)
    st_ref[0, 1:2] = jnp.sum(f3 * f3, axis=0, keepdims=True)


def _enc2_kernel(f3_ref, sc_ref, sh_ref, w4_ref, b4_ref, tok_ref):
    S = f3_ref.shape[0]
    K = GROUP_SIZE
    f3n = jnp.maximum(f3_ref[...] * sc_ref[...] + sh_ref[...], 0.0)
    f4 = jnp.dot(f3n, w4_ref[...], preferred_element_type=jnp.float32)
    f4 = f4 + b4_ref[...]
    tok_ref[...] = jnp.max(f4.reshape(S // K, K, f4.shape[1]), axis=1)


def _encoder_pallas(neighborhood, W1, b1, g1, be1, W2, b2, W3, b3, g3, be3,
                    W4, b4, interpret=False):
    bs, g, n, _ = neighborhood.shape
    S = bs * g * n
    eps = 1e-5
    X = neighborhood.reshape(S, 3)
    x3 = jnp.transpose(X, (1, 0))
    st = pl.pallas_call(
        _xstats_kernel,
        out_shape=jax.ShapeDtypeStruct((1, 16), jnp.float32),
        interpret=interpret,
    )(x3)[0]
    mean_x = st[0:3] / S
    m2 = jnp.array([[st[3], st[4], st[5]],
                    [st[4], st[6], st[7]],
                    [st[5], st[7], st[8]]]) / S
    cov = m2 - mean_x[:, None] * mean_x[None, :]
    mean1 = W1 @ mean_x + b1
    var1 = jnp.einsum('ci,ij,cj->c', W1, cov, W1)
    scale1 = g1 / jnp.sqrt(var1 + eps)
    bias1 = b1 * scale1 + (be1 - mean1 * scale1)
    W1p = W1.T * scale1[None, :]

    nblk = S // _SBLK
    f3, st3 = pl.pallas_call(
        _enc1_kernel,
        grid=(nblk,),
        in_specs=[
            pl.BlockSpec((_SBLK, 3), lambda i: (i, 0)),
            pl.BlockSpec((3, 128), lambda i: (0, 0)),
            pl.BlockSpec((1, 128), lambda i: (0, 0)),
            pl.BlockSpec((128, 256), lambda i: (0, 0)),
            pl.BlockSpec((1, 256), lambda i: (0, 0)),
            pl.BlockSpec((512, 512), lambda i: (0, 0)),
            pl.BlockSpec((1, 512), lambda i: (0, 0)),
        ],
        out_specs=[
            pl.BlockSpec((_SBLK, 512), lambda i: (i, 0)),
            pl.BlockSpec((1, 2, 512), lambda i: (i, 0, 0)),
        ],
        out_shape=[
            jax.ShapeDtypeStruct((S, 512), jnp.float32),
            jax.ShapeDtypeStruct((nblk, 2, 512), jnp.float32),
        ],
        compiler_params=pltpu.CompilerParams(
            dimension_semantics=("parallel",)),
        interpret=interpret,
    )(X, W1p, bias1.reshape(1, 128), W2.T, b2.reshape(1, 256),
      W3.T, b3.reshape(1, 512))
    ssum = jnp.sum(st3, axis=0)
    mean3 = ssum[0] / S
    var3 = ssum[1] / S - mean3 * mean3
    scale3 = g3 / jnp.sqrt(var3 + eps)
    shift3 = be3 - mean3 * scale3

    tokens = pl.pallas_call(
        _enc2_kernel,
        grid=(nblk,),
        in_specs=[
            pl.BlockSpec((_SBLK, 512), lambda i: (i, 0)),
            pl.BlockSpec((1, 512), lambda i: (0, 0)),
            pl.BlockSpec((1, 512), lambda i: (0, 0)),
            pl.BlockSpec((512, ENC_CH), lambda i: (0, 0)),
            pl.BlockSpec((1, ENC_CH), lambda i: (0, 0)),
        ],
        out_specs=pl.BlockSpec((_SBLK // GROUP_SIZE, ENC_CH), lambda i: (i, 0)),
        out_shape=jax.ShapeDtypeStruct((S // GROUP_SIZE, ENC_CH), jnp.float32),
        compiler_params=pltpu.CompilerParams(
            dimension_semantics=("parallel",)),
        interpret=interpret,
    )(f3, scale3.reshape(1, 512), shift3.reshape(1, 512),
      W4.T, b4.reshape(1, ENC_CH))
    return tokens.reshape(bs, g, ENC_CH)


def _index_points(points, idx):
    return jax.vmap(lambda p, i: p[i])(points, idx)


def _square_distance(src, dst):
    d = -2.0 * jnp.einsum('bsc,bnc->bsn', src, dst)
    d = d + jnp.sum(src ** 2, -1)[:, :, None]
    d = d + jnp.sum(dst ** 2, -1)[:, None, :]
    return d


def _conv1(x, W, b):
    return jnp.einsum('oi,bik->bok', W, x) + b[None, :, None]


def _batchnorm(x, gamma, beta, eps=1e-5):
    mean = jnp.mean(x, axis=(0, 2), keepdims=True)
    var = jnp.var(x, axis=(0, 2), keepdims=True)
    xn = (x - mean) / jnp.sqrt(var + eps)
    return gamma[None, :, None] * xn + beta[None, :, None]


def _sub_kernel(nb_ref, c_ref, o_ref):
    o_ref[...] = nb_ref[...] - c_ref[...]


def _encoder(neighborhood, W1, b1, g1, be1, W2, b2, W3, b3, g3, be3, W4, b4):
    bs, g, n, _ = neighborhood.shape
    pg = neighborhood.reshape(bs * g, n, IN_CH).transpose(0, 2, 1)
    f = _conv1(pg, W1, b1)
    f = jax.nn.relu(_batchnorm(f, g1, be1))
    f = _conv1(f, W2, b2)
    fg = jnp.max(f, axis=2, keepdims=True)
    f = jnp.concatenate([jnp.broadcast_to(fg, (bs * g, 256, n)), f], axis=1)
    f = _conv1(f, W3, b3)
    f = jax.nn.relu(_batchnorm(f, g3, be3))
    f = _conv1(f, W4, b4)
    fg = jnp.max(f, axis=2)
    return fg.reshape(bs, g, ENC_CH)


def kernel(xyz, W1, b1, g1, be1, W2, b2, W3, b3, g3, be3, W4, b4):
    B, N, _ = xyz.shape
    center = _fps_centers(xyz, NUM_GROUP)
    neighborhood = _knn_neighborhood(xyz, center)
    tokens = _encoder_pallas(neighborhood, W1, b1, g1, be1, W2, b2,
                             W3, b3, g3, be3, W4, b4)
    return tokens
